# baseline (speedup 1.0000x reference)
import functools

import jax
import jax.numpy as jnp
from jax import lax
from jax.experimental import pallas as pl
from jax.experimental.pallas import tpu as pltpu
from jax.experimental.pallas import tpu_sc as plsc

TOPK = 5
NROWS = 128
N = 32768
NC = 2
NS = 16
NW = NC * NS
SCROWS = 64
TCR = NROWS - SCROWS
ROWS_PER_W = SCROWS // NW
L = 16
U = 8
CH = L * U
NIT = N // CH
NPH = 2 * ROWS_PER_W
CBLK = 2048
NBLK = N // CBLK
F32_MIN = jnp.finfo(jnp.float32).min


def _sc_body(yhat_hbm, y_hbm, out_hbm, buf0, buf1, buf2, obuf,
             sem0, sem1, sem2):
  bufs = (buf0, buf1, buf2)
  sems = (sem0, sem1, sem2)
  wid = lax.axis_index("s") * NC + lax.axis_index("c")
  base_row = wid * ROWS_PER_W
  iota = lax.iota(jnp.int32, L)
  iotas = [iota + u * L for u in range(U)]

  def copy(p):
    r = base_row + p // 2
    src = yhat_hbm.at[r] if p % 2 else y_hbm.at[pl.ds(r * N, N)]
    return pltpu.make_async_copy(src, bufs[p % 3], sems[p % 3])

  copy(0).start()
  copy(1).start()
  hits = jnp.float32(0.0)
  t = jnp.int32(0)
  for p in range(NPH):
    if p + 2 < NPH:
      copy(p + 2).start()
    copy(p).wait()
    buf = bufs[p % 3]

    if p % 2 == 0:
      def amax_body(j, c, buf=buf):
        bvs, bjs = c[:U], c[U:]
        base = j * CH
        jv = jnp.full((L,), j, jnp.int32)
        nbvs, nbjs = [], []
        for u in range(U):
          x = buf[pl.ds(base + u * L, L)]
          pgt = x > bvs[u]
          nbvs.append(jnp.where(pgt, x, bvs[u]))
          nbjs.append(jnp.where(pgt, jv, bjs[u]))
        return tuple(nbvs) + tuple(nbjs)

      c = lax.fori_loop(
          0, NIT, amax_body,
          tuple(jnp.full((L,), F32_MIN, jnp.float32) for _ in range(U))
          + tuple(jnp.zeros((L,), jnp.int32) for _ in range(U)))
      bvs, bjs = c[:U], c[U:]
      m = jnp.max(bvs[0])
      for u in range(1, U):
        m = jnp.maximum(m, jnp.max(bvs[u]))
      t = jnp.int32(N)
      for u in range(U):
        idxs = bjs[u] * CH + iotas[u]
        t = jnp.minimum(t, jnp.min(jnp.where(bvs[u] == m, idxs,
                                             jnp.int32(N))))
    else:
      tvec = jnp.full((L,), t, jnp.int32)
      v = plsc.load_gather(buf, [tvec])
      jb = t // CH

      def pre_body(j, c, buf=buf, v=v):
        base = j * CH
        out = []
        for u in range(U):
          x = buf[pl.ds(base + u * L, L)]
          out.append(c[u] + (x >= v).astype(jnp.int32))
        return tuple(out)

      c = lax.fori_loop(0, jb, pre_body,
                        tuple(jnp.zeros((L,), jnp.int32) for _ in range(U)))

      def suf_body(j, c, buf=buf, v=v):
        base = j * CH
        out = []
        for u in range(U):
          x = buf[pl.ds(base + u * L, L)]
          out.append(c[u] + (x > v).astype(jnp.int32))
        return tuple(out)

      c = lax.fori_loop(jb + 1, NIT, suf_body, c)

      base = jb * CH
      rank = jnp.int32(0)
      for u in range(U):
        x = buf[pl.ds(base + u * L, L)]
        idx = iotas[u] + base
        bc = (x > v) | ((x == v) & (idx < tvec))
        rank = rank + jnp.sum(bc.astype(jnp.int32) + c[u])
      hits = hits + jnp.where(rank < TOPK, jnp.float32(1.0), jnp.float32(0.0))

  obuf[...] = jnp.full((L,), hits, jnp.float32)
  pltpu.sync_copy(obuf, out_hbm.at[wid])


TCB = 8


def _tc_argmax(y_ref, o_ref):
  x = jnp.reshape(y_ref[...], (TCB, N // 128, 128))
  m2 = jnp.max(x, axis=1)
  m = jnp.max(m2, axis=1, keepdims=True)
  ci = lax.broadcasted_iota(jnp.int32, (TCB, N // 128, 128), 1)
  cl = lax.broadcasted_iota(jnp.int32, (TCB, N // 128, 128), 2)
  cols = ci * 128 + cl
  idx3 = jnp.where(x == m[:, :, None], cols, jnp.int32(N))
  i2 = jnp.min(idx3, axis=1)
  t = jnp.min(i2, axis=1, keepdims=True)
  o_ref[...] = jnp.broadcast_to(t[:, :, None], (TCB, 1, 128))


def _tc_count(yhat_ref, t_ref, o_ref):
  g = pl.program_id(0)
  x = yhat_ref[...]
  t = t_ref[...][:, 0, 0:1]
  cols = lax.broadcasted_iota(jnp.int32, (TCB, N), 1)
  v = jnp.sum(jnp.where(cols == t, x, jnp.float32(0.0)), axis=1,
              keepdims=True)
  bc = (x > v) | ((x == v) & (cols < t))
  cnt = jnp.sum(bc.astype(jnp.int32), axis=1, keepdims=True)
  hits = jnp.sum((cnt < TOPK).astype(jnp.float32))
  prev = jnp.where(g == 0, jnp.float32(0.0), o_ref[0, 0])
  o_ref[...] = jnp.full((1, 1), prev + hits, jnp.float32)


def _tc_merge(p_ref, tc_ref, o_ref):
  total = jnp.sum(p_ref[...]) * (1.0 / L) + tc_ref[0, 0]
  o_ref[...] = jnp.full((1, 1), (1.0 - total / NROWS) * 100.0, jnp.float32)


@jax.jit
def kernel(yhat, y):
  y1d = jnp.reshape(y, (NROWS * N,))
  yview = jnp.reshape(y, (NROWS * N // 128, 128))

  targets = pl.pallas_call(
      _tc_argmax,
      grid=(TCR // TCB,),
      in_specs=[pl.BlockSpec((TCB * (N // 128), 128),
                             lambda g: (SCROWS // TCB + g, 0))],
      out_specs=pl.BlockSpec((TCB, 1, 128), lambda g: (g, 0, 0)),
      out_shape=jax.ShapeDtypeStruct((TCR, 1, 128), jnp.int32),
  )(yview)

  tc_hits = pl.pallas_call(
      _tc_count,
      grid=(TCR // TCB,),
      in_specs=[
          pl.BlockSpec((TCB, N), lambda g: (SCROWS // TCB + g, 0)),
          pl.BlockSpec((TCB, 1, 128), lambda g: (g, 0, 0)),
      ],
      out_specs=pl.BlockSpec((1, 1), lambda g: (0, 0)),
      out_shape=jax.ShapeDtypeStruct((1, 1), jnp.float32),
  )(yhat, targets)

  mesh = plsc.VectorSubcoreMesh(core_axis_name="c", subcore_axis_name="s")
  sc_k = functools.partial(
      pl.kernel,
      mesh=mesh,
      compiler_params=pltpu.CompilerParams(needs_layout_passes=False,
                                           use_tc_tiling_on_sc=True),
      out_type=jax.ShapeDtypeStruct((NW, L), jnp.float32),
      scratch_types=[
          pltpu.VMEM((N,), jnp.float32),
          pltpu.VMEM((N,), jnp.float32),
          pltpu.VMEM((N,), jnp.float32),
          pltpu.VMEM((L,), jnp.float32),
          pltpu.SemaphoreType.DMA,
          pltpu.SemaphoreType.DMA,
          pltpu.SemaphoreType.DMA,
      ],
  )(_sc_body)
  partial_hits = sc_k(yhat, y1d)

  err = pl.pallas_call(
      _tc_merge,
      out_shape=jax.ShapeDtypeStruct((1, 1), jnp.float32),
  )(partial_hits, tc_hits)
  return jnp.reshape(err, ())


# speedup vs baseline: 1.0323x; 1.0323x over previous
import functools

import jax
import jax.numpy as jnp
from jax import lax
from jax.experimental import pallas as pl
from jax.experimental.pallas import tpu as pltpu
from jax.experimental.pallas import tpu_sc as plsc

TOPK = 5
NROWS = 128
N = 32768
NC = 2
NS = 16
NW = NC * NS
SCROWS = 96
TCR = NROWS - SCROWS
ROWS_PER_W = SCROWS // NW
L = 16
U = 8
CH = L * U
NIT = N // CH
NPH = 2 * ROWS_PER_W
CBLK = 2048
NBLK = N // CBLK
F32_MIN = jnp.finfo(jnp.float32).min


def _sc_body(yhat_hbm, y_hbm, out_hbm, buf0, buf1, buf2, obuf,
             sem0, sem1, sem2):
  bufs = (buf0, buf1, buf2)
  sems = (sem0, sem1, sem2)
  wid = lax.axis_index("s") * NC + lax.axis_index("c")
  base_row = wid * ROWS_PER_W
  iota = lax.iota(jnp.int32, L)
  iotas = [iota + u * L for u in range(U)]

  def copy(p):
    r = base_row + p // 2
    src = yhat_hbm.at[r] if p % 2 else y_hbm.at[pl.ds(r * N, N)]
    return pltpu.make_async_copy(src, bufs[p % 3], sems[p % 3])

  copy(0).start()
  copy(1).start()
  hits = jnp.float32(0.0)
  t = jnp.int32(0)
  for p in range(NPH):
    if p + 2 < NPH:
      copy(p + 2).start()
    copy(p).wait()
    buf = bufs[p % 3]

    if p % 2 == 0:
      def amax_body(j, c, buf=buf):
        bvs, bjs = c[:U], c[U:]
        base = j * CH
        jv = jnp.full((L,), j, jnp.int32)
        nbvs, nbjs = [], []
        for u in range(U):
          x = buf[pl.ds(base + u * L, L)]
          pgt = x > bvs[u]
          nbvs.append(jnp.where(pgt, x, bvs[u]))
          nbjs.append(jnp.where(pgt, jv, bjs[u]))
        return tuple(nbvs) + tuple(nbjs)

      c = lax.fori_loop(
          0, NIT, amax_body,
          tuple(jnp.full((L,), F32_MIN, jnp.float32) for _ in range(U))
          + tuple(jnp.zeros((L,), jnp.int32) for _ in range(U)))
      bvs, bjs = c[:U], c[U:]
      m = jnp.max(bvs[0])
      for u in range(1, U):
        m = jnp.maximum(m, jnp.max(bvs[u]))
      t = jnp.int32(N)
      for u in range(U):
        idxs = bjs[u] * CH + iotas[u]
        t = jnp.minimum(t, jnp.min(jnp.where(bvs[u] == m, idxs,
                                             jnp.int32(N))))
    else:
      tvec = jnp.full((L,), t, jnp.int32)
      v = plsc.load_gather(buf, [tvec])
      jb = t // CH

      def pre_body(j, c, buf=buf, v=v):
        base = j * CH
        out = []
        for u in range(U):
          x = buf[pl.ds(base + u * L, L)]
          out.append(c[u] + (x >= v).astype(jnp.int32))
        return tuple(out)

      c = lax.fori_loop(0, jb, pre_body,
                        tuple(jnp.zeros((L,), jnp.int32) for _ in range(U)))

      def suf_body(j, c, buf=buf, v=v):
        base = j * CH
        out = []
        for u in range(U):
          x = buf[pl.ds(base + u * L, L)]
          out.append(c[u] + (x > v).astype(jnp.int32))
        return tuple(out)

      c = lax.fori_loop(jb + 1, NIT, suf_body, c)

      base = jb * CH
      rank = jnp.int32(0)
      for u in range(U):
        x = buf[pl.ds(base + u * L, L)]
        idx = iotas[u] + base
        bc = (x > v) | ((x == v) & (idx < tvec))
        rank = rank + jnp.sum(bc.astype(jnp.int32) + c[u])
      hits = hits + jnp.where(rank < TOPK, jnp.float32(1.0), jnp.float32(0.0))

  obuf[...] = jnp.full((L,), hits, jnp.float32)
  pltpu.sync_copy(obuf, out_hbm.at[wid])


def _tc_argmax(y_ref, o_ref):
  x = jnp.reshape(y_ref[...], (TCR, N // 128, 128))
  m2 = jnp.max(x, axis=1)
  m = jnp.max(m2, axis=1, keepdims=True)
  ci = lax.broadcasted_iota(jnp.int32, (TCR, N // 128, 128), 1)
  cl = lax.broadcasted_iota(jnp.int32, (TCR, N // 128, 128), 2)
  cols = ci * 128 + cl
  idx3 = jnp.where(x == m[:, :, None], cols, jnp.int32(N))
  i2 = jnp.min(idx3, axis=1)
  t = jnp.min(i2, axis=1, keepdims=True)
  o_ref[...] = jnp.broadcast_to(t[:, :, None], (TCR, 1, 128))


def _tc_count(yhat_ref, t_ref, o_ref):
  x = yhat_ref[...]
  t = t_ref[...][:, 0, 0:1]
  cols = lax.broadcasted_iota(jnp.int32, (TCR, N), 1)
  v = jnp.sum(jnp.where(cols == t, x, jnp.float32(0.0)), axis=1,
              keepdims=True)
  bc = (x > v) | ((x == v) & (cols < t))
  cnt = jnp.sum(bc.astype(jnp.int32), axis=1, keepdims=True)
  hits = jnp.sum((cnt < TOPK).astype(jnp.float32))
  o_ref[...] = jnp.full((1, 1), hits, jnp.float32)


def _tc_merge(p_ref, tc_ref, o_ref):
  total = jnp.sum(p_ref[...]) * (1.0 / L) + tc_ref[0, 0]
  o_ref[...] = jnp.full((1, 1), (1.0 - total / NROWS) * 100.0, jnp.float32)


@jax.jit
def kernel(yhat, y):
  y1d = jnp.reshape(y, (NROWS * N,))
  yview = jnp.reshape(y, (NROWS * N // 128, 128))

  targets = pl.pallas_call(
      _tc_argmax,
      grid=(1,),
      in_specs=[pl.BlockSpec((TCR * (N // 128), 128),
                             lambda i: (SCROWS // TCR, 0))],
      out_specs=pl.BlockSpec((TCR, 1, 128), lambda i: (0, 0, 0)),
      out_shape=jax.ShapeDtypeStruct((TCR, 1, 128), jnp.int32),
  )(yview)

  tc_hits = pl.pallas_call(
      _tc_count,
      grid=(1,),
      in_specs=[
          pl.BlockSpec((TCR, N), lambda i: (SCROWS // TCR, 0)),
          pl.BlockSpec((TCR, 1, 128), lambda i: (0, 0, 0)),
      ],
      out_specs=pl.BlockSpec((1, 1), lambda i: (0, 0)),
      out_shape=jax.ShapeDtypeStruct((1, 1), jnp.float32),
  )(yhat, targets)

  mesh = plsc.VectorSubcoreMesh(core_axis_name="c", subcore_axis_name="s")
  sc_k = functools.partial(
      pl.kernel,
      mesh=mesh,
      compiler_params=pltpu.CompilerParams(needs_layout_passes=False,
                                           use_tc_tiling_on_sc=True),
      out_type=jax.ShapeDtypeStruct((NW, L), jnp.float32),
      scratch_types=[
          pltpu.VMEM((N,), jnp.float32),
          pltpu.VMEM((N,), jnp.float32),
          pltpu.VMEM((N,), jnp.float32),
          pltpu.VMEM((L,), jnp.float32),
          pltpu.SemaphoreType.DMA,
          pltpu.SemaphoreType.DMA,
          pltpu.SemaphoreType.DMA,
      ],
  )(_sc_body)
  partial_hits = sc_k(yhat, y1d)

  err = pl.pallas_call(
      _tc_merge,
      out_shape=jax.ShapeDtypeStruct((1, 1), jnp.float32),
  )(partial_hits, tc_hits)
  return jnp.reshape(err, ())


# speedup vs baseline: 1.0333x; 1.0010x over previous
import functools

import jax
import jax.numpy as jnp
from jax import lax
from jax.experimental import pallas as pl
from jax.experimental.pallas import tpu as pltpu
from jax.experimental.pallas import tpu_sc as plsc

TOPK = 5
NROWS = 128
N = 32768
NC = 2
NS = 16
NW = NC * NS
SCROWS = 96
TCR = NROWS - SCROWS
ROWS_PER_W = SCROWS // NW
L = 16
U = 8
CH = L * U
NIT = N // CH
NPH = 2 * ROWS_PER_W
F32_MIN = jnp.finfo(jnp.float32).min


def _sc_body(yhat_hbm, y_hbm, out_hbm, buf0, buf1, buf2, obuf,
             sem0, sem1, sem2):
  bufs = (buf0, buf1, buf2)
  sems = (sem0, sem1, sem2)
  wid = lax.axis_index("s") * NC + lax.axis_index("c")
  base_row = wid * ROWS_PER_W
  iota = lax.iota(jnp.int32, L)
  iotas = [iota + u * L for u in range(U)]

  def copy(p):
    r = base_row + p // 2
    src = yhat_hbm.at[r] if p % 2 else y_hbm.at[pl.ds(r * N, N)]
    return pltpu.make_async_copy(src, bufs[p % 3], sems[p % 3])

  copy(0).start()
  copy(1).start()
  hits = jnp.float32(0.0)
  t = jnp.int32(0)
  for p in range(NPH):
    if p + 2 < NPH:
      copy(p + 2).start()
    copy(p).wait()
    buf = bufs[p % 3]

    if p % 2 == 0:
      def amax_body(j, c, buf=buf):
        bvs, bjs = c[:U], c[U:]
        base = j * CH
        jv = jnp.full((L,), j, jnp.int32)
        nbvs, nbjs = [], []
        for u in range(U):
          x = buf[pl.ds(base + u * L, L)]
          pgt = x > bvs[u]
          nbvs.append(jnp.where(pgt, x, bvs[u]))
          nbjs.append(jnp.where(pgt, jv, bjs[u]))
        return tuple(nbvs) + tuple(nbjs)

      c = lax.fori_loop(
          0, NIT, amax_body,
          tuple(jnp.full((L,), F32_MIN, jnp.float32) for _ in range(U))
          + tuple(jnp.zeros((L,), jnp.int32) for _ in range(U)))
      bvs, bjs = c[:U], c[U:]
      m = jnp.max(bvs[0])
      for u in range(1, U):
        m = jnp.maximum(m, jnp.max(bvs[u]))
      t = jnp.int32(N)
      for u in range(U):
        idxs = bjs[u] * CH + iotas[u]
        t = jnp.minimum(t, jnp.min(jnp.where(bvs[u] == m, idxs,
                                             jnp.int32(N))))
    else:
      tvec = jnp.full((L,), t, jnp.int32)
      v = plsc.load_gather(buf, [tvec])
      jb = t // CH

      def pre_body(j, c, buf=buf, v=v):
        base = j * CH
        out = []
        for u in range(U):
          x = buf[pl.ds(base + u * L, L)]
          out.append(c[u] + (x >= v).astype(jnp.int32))
        return tuple(out)

      c = lax.fori_loop(0, jb, pre_body,
                        tuple(jnp.zeros((L,), jnp.int32) for _ in range(U)))

      def suf_body(j, c, buf=buf, v=v):
        base = j * CH
        out = []
        for u in range(U):
          x = buf[pl.ds(base + u * L, L)]
          out.append(c[u] + (x > v).astype(jnp.int32))
        return tuple(out)

      c = lax.fori_loop(jb + 1, NIT, suf_body, c)

      base = jb * CH
      rank = jnp.int32(0)
      for u in range(U):
        x = buf[pl.ds(base + u * L, L)]
        idx = iotas[u] + base
        bc = (x > v) | ((x == v) & (idx < tvec))
        rank = rank + jnp.sum(bc.astype(jnp.int32) + c[u])
      hits = hits + jnp.where(rank < TOPK, jnp.float32(1.0), jnp.float32(0.0))

  obuf[...] = jnp.full((L,), hits, jnp.float32)
  pltpu.sync_copy(obuf, out_hbm.at[wid])


def _tc_argmax(y_ref, o_ref):
  x = jnp.reshape(y_ref[...], (TCR, N // 128, 128))
  m2 = jnp.max(x, axis=1)
  m = jnp.max(m2, axis=1, keepdims=True)
  ci = lax.broadcasted_iota(jnp.int32, (TCR, N // 128, 128), 1)
  cl = lax.broadcasted_iota(jnp.int32, (TCR, N // 128, 128), 2)
  cols = ci * 128 + cl
  idx3 = jnp.where(x == m[:, :, None], cols, jnp.int32(N))
  i2 = jnp.min(idx3, axis=1)
  t = jnp.min(i2, axis=1, keepdims=True)
  o_ref[...] = jnp.broadcast_to(t[:, :, None], (TCR, 1, 128))


def _tc_count(yhat_ref, t_ref, o_ref):
  x = yhat_ref[...]
  t = t_ref[...][:, 0, 0:1]
  cols = lax.broadcasted_iota(jnp.int32, (TCR, N), 1)
  v = jnp.sum(jnp.where(cols == t, x, jnp.float32(0.0)), axis=1,
              keepdims=True)
  bc = (x > v) | ((x == v) & (cols < t))
  cnt = jnp.sum(bc.astype(jnp.int32), axis=1, keepdims=True)
  hits = jnp.sum((cnt < TOPK).astype(jnp.float32))
  o_ref[...] = jnp.full((1, 1), hits, jnp.float32)


def _tc_merge(p_ref, tc_ref, o_ref):
  total = jnp.sum(p_ref[...]) * (1.0 / L) + tc_ref[0, 0]
  o_ref[...] = jnp.full((1, 1), (1.0 - total / NROWS) * 100.0, jnp.float32)


@jax.jit
def kernel(yhat, y):
  y1d = jnp.reshape(y, (NROWS * N,))
  yview = jnp.reshape(y, (NROWS * N // 128, 128))

  targets = pl.pallas_call(
      _tc_argmax,
      grid=(1,),
      in_specs=[pl.BlockSpec((TCR * (N // 128), 128),
                             lambda i: (SCROWS // TCR, 0))],
      out_specs=pl.BlockSpec((TCR, 1, 128), lambda i: (0, 0, 0)),
      out_shape=jax.ShapeDtypeStruct((TCR, 1, 128), jnp.int32),
  )(yview)

  tc_hits = pl.pallas_call(
      _tc_count,
      grid=(1,),
      in_specs=[
          pl.BlockSpec((TCR, N), lambda i: (SCROWS // TCR, 0)),
          pl.BlockSpec((TCR, 1, 128), lambda i: (0, 0, 0)),
      ],
      out_specs=pl.BlockSpec((1, 1), lambda i: (0, 0)),
      out_shape=jax.ShapeDtypeStruct((1, 1), jnp.float32),
  )(yhat, targets)

  mesh = plsc.VectorSubcoreMesh(core_axis_name="c", subcore_axis_name="s")
  sc_k = functools.partial(
      pl.kernel,
      mesh=mesh,
      compiler_params=pltpu.CompilerParams(needs_layout_passes=False,
                                           use_tc_tiling_on_sc=True),
      out_type=jax.ShapeDtypeStruct((NW, L), jnp.float32),
      scratch_types=[
          pltpu.VMEM((N,), jnp.float32),
          pltpu.VMEM((N,), jnp.float32),
          pltpu.VMEM((N,), jnp.float32),
          pltpu.VMEM((L,), jnp.float32),
          pltpu.SemaphoreType.DMA,
          pltpu.SemaphoreType.DMA,
          pltpu.SemaphoreType.DMA,
      ],
  )(_sc_body)
  partial_hits = sc_k(yhat, y1d)

  err = pl.pallas_call(
      _tc_merge,
      out_shape=jax.ShapeDtypeStruct((1, 1), jnp.float32),
  )(partial_hits, tc_hits)
  return jnp.reshape(err, ())
